# baseline (device time: 13186 ns/iter reference)
import jax
import jax.numpy as jnp
from jax import lax
from jax.experimental import pallas as pl
from jax.experimental.pallas import tpu as pltpu

N_DEV = 4
C = 4


def kernel(partial, resid, gamma):
    m, d = resid.shape
    h = m // 2
    q = h // C

    def slot(rnd, hh, c, sr):
        return ((rnd * 2 + hh) * C + c) * 2 + sr

    def sem_idx(rnd, hh, c):
        return (rnd * 2 + hh) * C + c

    def body(
        x_ref, resid_ref, gamma_ref, out_ref,
        xv, rv, gv, comm, send_sems, recv_sems, local_sems,
    ):
        i = lax.axis_index("i")
        px = N_DEV - 1 - i
        py = i + 1 - 2 * (i % 2)
        partner = [[px, py], [py, px]]

        def exchange(rnd, hh, c):
            k = sem_idx(rnd, hh, c)
            return pltpu.make_async_remote_copy(
                src_ref=comm.at[slot(rnd, hh, c, 0)],
                dst_ref=comm.at[slot(rnd, hh, c, 1)],
                send_sem=send_sems.at[k],
                recv_sem=recv_sems.at[k],
                device_id=(partner[rnd][hh],),
                device_id_type=pl.DeviceIdType.MESH,
            )

        cp_x = {}
        for hh in range(2):
            for c in range(C):
                cp = pltpu.make_async_copy(
                    x_ref.at[0, pl.ds(hh * h + c * q, q), :],
                    xv.at[hh, pl.ds(c * q, q), :],
                    local_sems.at[hh * C + c],
                )
                cp.start()
                cp_x[hh, c] = cp
        cp_r = pltpu.make_async_copy(resid_ref, rv, local_sems.at[2 * C])
        cp_g = pltpu.make_async_copy(gamma_ref, gv, local_sems.at[2 * C + 1])
        cp_r.start()
        cp_g.start()

        import os as _os
        _local = _os.environ.get("KERNEL_LOCAL_ONLY") == "1"

        for hh in range(2):
            cp_x[hh, 0].wait()
            comm[slot(0, hh, 0, 0), :, :] = xv[hh, 0:q, :].astype(jnp.bfloat16)

        barrier_sem = pltpu.get_barrier_semaphore()
        for nbr in (px, py):
            pl.semaphore_signal(
                barrier_sem, inc=1,
                device_id=(nbr,), device_id_type=pl.DeviceIdType.MESH,
            )
        pl.semaphore_wait(barrier_sem, 2)

        r1 = {}
        for c in range(C):
            for hh in range(2):
                if c > 0:
                    cp_x[hh, c].wait()
                    comm[slot(0, hh, c, 0), :, :] = xv[
                        hh, c * q : (c + 1) * q, :
                    ].astype(jnp.bfloat16)
                r1[hh, c] = exchange(0, hh, c)
                if not _local:
                    r1[hh, c].start()

        acc = {}
        r2 = {}
        for c in range(C):
            for hh in range(2):
                if not _local:
                    r1[hh, c].wait()
                a = xv[hh, c * q : (c + 1) * q, :] + comm[
                    slot(0, hh, c, 1), :, :
                ].astype(jnp.float32)
                acc[hh, c] = a
                comm[slot(1, hh, c, 0), :, :] = a.astype(jnp.bfloat16)
                r2[hh, c] = exchange(1, hh, c)
                if not _local:
                    r2[hh, c].start()

        cp_r.wait()
        cp_g.wait()
        gam = jnp.reshape(gv[...], (1, d))

        for c in range(C):
            for hh in range(2):
                if not _local:
                    r2[hh, c].wait()
                row0 = hh * h + c * q
                y = (
                    acc[hh, c]
                    + comm[slot(1, hh, c, 1), :, :].astype(jnp.float32)
                    + rv[row0 : row0 + q, :]
                )
                rms = jnp.sqrt(jnp.mean(y * y, axis=-1, keepdims=True) + 1e-6)
                out_ref[row0 : row0 + q, :] = y / rms * gam

    return pl.pallas_call(
        body,
        out_shape=jax.ShapeDtypeStruct((m, d), jnp.float32),
        in_specs=[
            pl.BlockSpec(memory_space=pl.ANY),
            pl.BlockSpec(memory_space=pl.ANY),
            pl.BlockSpec(memory_space=pl.ANY),
        ],
        out_specs=pl.BlockSpec(memory_space=pltpu.VMEM),
        scratch_shapes=[
            pltpu.VMEM((2, h, d), jnp.float32),
            pltpu.VMEM((m, d), jnp.float32),
            pltpu.VMEM((d,), jnp.float32),
            pltpu.VMEM((4 * 2 * C, q, d), jnp.bfloat16),
            pltpu.SemaphoreType.DMA((2 * 2 * C,)),
            pltpu.SemaphoreType.DMA((2 * 2 * C,)),
            pltpu.SemaphoreType.DMA((2 * C + 2,)),
        ],
        compiler_params=pltpu.CompilerParams(collective_id=0),
    )(
        pltpu.with_memory_space_constraint(partial, pltpu.MemorySpace.HBM),
        pltpu.with_memory_space_constraint(resid, pltpu.MemorySpace.HBM),
        pltpu.with_memory_space_constraint(gamma, pltpu.MemorySpace.HBM),
    )


# device time: 12033 ns/iter; 1.0958x vs baseline; 1.0958x over previous
import jax
import jax.numpy as jnp
from jax import lax
from jax.experimental import pallas as pl
from jax.experimental.pallas import tpu as pltpu

N_DEV = 4
C = 4


def kernel(partial, resid, gamma):
    m, d = resid.shape
    h = m // 2
    q = h // C

    def slot(rnd, hh, c, sr):
        return ((rnd * 2 + hh) * C + c) * 2 + sr

    def sem_idx(rnd, hh, c):
        return (rnd * 2 + hh) * C + c

    def body(
        x_ref, resid_ref, gamma_ref, out_ref,
        xv, rv, gv, comm, send_sems, recv_sems, local_sems,
    ):
        i = lax.axis_index("i")
        px = N_DEV - 1 - i
        py = i + 1 - 2 * (i % 2)
        partner = [[px, py], [py, px]]

        def exchange(rnd, hh, c):
            k = sem_idx(rnd, hh, c)
            return pltpu.make_async_remote_copy(
                src_ref=comm.at[slot(rnd, hh, c, 0)],
                dst_ref=comm.at[slot(rnd, hh, c, 1)],
                send_sem=send_sems.at[k],
                recv_sem=recv_sems.at[k],
                device_id=(partner[rnd][hh],),
                device_id_type=pl.DeviceIdType.MESH,
            )

        cp_x = {}
        for hh in range(2):
            for c in range(C):
                cp = pltpu.make_async_copy(
                    x_ref.at[0, pl.ds(hh * h + c * q, q), :],
                    xv.at[hh, pl.ds(c * q, q), :],
                    local_sems.at[hh * C + c],
                )
                cp.start()
                cp_x[hh, c] = cp
        cp_r = pltpu.make_async_copy(resid_ref, rv, local_sems.at[2 * C])
        cp_g = pltpu.make_async_copy(gamma_ref, gv, local_sems.at[2 * C + 1])
        cp_r.start()
        cp_g.start()

        barrier_sem = pltpu.get_barrier_semaphore()
        for nbr in (px, py):
            pl.semaphore_signal(
                barrier_sem, inc=1,
                device_id=(nbr,), device_id_type=pl.DeviceIdType.MESH,
            )
        pl.semaphore_wait(barrier_sem, 2)

        r1 = {}
        for c in range(C):
            for hh in range(2):
                cp_x[hh, c].wait()
                comm[slot(0, hh, c, 0), :, :] = xv[
                    hh, c * q : (c + 1) * q, :
                ].astype(jnp.bfloat16)
                r1[hh, c] = exchange(0, hh, c)
                r1[hh, c].start()

        acc = {}
        r2 = {}
        for c in range(C):
            for hh in range(2):
                r1[hh, c].wait()
                a = xv[hh, c * q : (c + 1) * q, :] + comm[
                    slot(0, hh, c, 1), :, :
                ].astype(jnp.float32)
                acc[hh, c] = a
                comm[slot(1, hh, c, 0), :, :] = a.astype(jnp.bfloat16)
                r2[hh, c] = exchange(1, hh, c)
                r2[hh, c].start()

        cp_r.wait()
        cp_g.wait()
        gam = jnp.reshape(gv[...], (1, d))

        for c in range(C):
            for hh in range(2):
                r2[hh, c].wait()
                row0 = hh * h + c * q
                y = (
                    acc[hh, c]
                    + comm[slot(1, hh, c, 1), :, :].astype(jnp.float32)
                    + rv[row0 : row0 + q, :]
                )
                rms = jnp.sqrt(jnp.mean(y * y, axis=-1, keepdims=True) + 1e-6)
                out_ref[row0 : row0 + q, :] = y / rms * gam

    return pl.pallas_call(
        body,
        out_shape=jax.ShapeDtypeStruct((m, d), jnp.float32),
        in_specs=[
            pl.BlockSpec(memory_space=pl.ANY),
            pl.BlockSpec(memory_space=pl.ANY),
            pl.BlockSpec(memory_space=pl.ANY),
        ],
        out_specs=pl.BlockSpec(memory_space=pltpu.VMEM),
        scratch_shapes=[
            pltpu.VMEM((2, h, d), jnp.float32),
            pltpu.VMEM((m, d), jnp.float32),
            pltpu.VMEM((d,), jnp.float32),
            pltpu.VMEM((4 * 2 * C, q, d), jnp.bfloat16),
            pltpu.SemaphoreType.DMA((2 * 2 * C,)),
            pltpu.SemaphoreType.DMA((2 * 2 * C,)),
            pltpu.SemaphoreType.DMA((2 * C + 2,)),
        ],
        compiler_params=pltpu.CompilerParams(collective_id=0),
    )(
        pltpu.with_memory_space_constraint(partial, pltpu.MemorySpace.HBM),
        pltpu.with_memory_space_constraint(resid, pltpu.MemorySpace.HBM),
        pltpu.with_memory_space_constraint(gamma, pltpu.MemorySpace.HBM),
    )


# device time: 12021 ns/iter; 1.0969x vs baseline; 1.0010x over previous
import jax
import jax.numpy as jnp
from jax import lax
from jax.experimental import pallas as pl
from jax.experimental.pallas import tpu as pltpu

N_DEV = 4
C = 4


def kernel(partial, resid, gamma):
    m, d = resid.shape
    h = m // 2
    q = h // C

    def slot(rnd, hh, c, sr):
        return ((rnd * 2 + hh) * C + c) * 2 + sr

    def sem_idx(rnd, hh, c):
        return (rnd * 2 + hh) * C + c

    def body(
        x_ref, resid_ref, gamma_ref, out_ref,
        xv, rv, gv, comm, send_sems, recv_sems, local_sems,
    ):
        i = lax.axis_index("i")
        px = N_DEV - 1 - i
        py = i + 1 - 2 * (i % 2)
        partner = [[px, py], [py, px]]

        def exchange(rnd, hh, c):
            k = sem_idx(rnd, hh, c)
            return pltpu.make_async_remote_copy(
                src_ref=comm.at[slot(rnd, hh, c, 0)],
                dst_ref=comm.at[slot(rnd, hh, c, 1)],
                send_sem=send_sems.at[k],
                recv_sem=recv_sems.at[k],
                device_id=(partner[rnd][hh],),
                device_id_type=pl.DeviceIdType.MESH,
            )

        barrier_sem = pltpu.get_barrier_semaphore()
        for nbr in (px, py):
            pl.semaphore_signal(
                barrier_sem, inc=1,
                device_id=(nbr,), device_id_type=pl.DeviceIdType.MESH,
            )

        cp_x = {}
        for hh in range(2):
            for c in range(C):
                cp = pltpu.make_async_copy(
                    x_ref.at[0, pl.ds(hh * h + c * q, q), :],
                    xv.at[hh, pl.ds(c * q, q), :],
                    local_sems.at[hh * C + c],
                )
                cp.start()
                cp_x[hh, c] = cp
        cp_r = pltpu.make_async_copy(resid_ref, rv, local_sems.at[2 * C])
        cp_g = pltpu.make_async_copy(gamma_ref, gv, local_sems.at[2 * C + 1])
        cp_r.start()
        cp_g.start()

        for hh in range(2):
            cp_x[hh, 0].wait()
            comm[slot(0, hh, 0, 0), :, :] = xv[hh, 0:q, :].astype(jnp.bfloat16)

        pl.semaphore_wait(barrier_sem, 2)

        r1 = {}
        for c in range(C):
            for hh in range(2):
                if c > 0:
                    cp_x[hh, c].wait()
                    comm[slot(0, hh, c, 0), :, :] = xv[
                        hh, c * q : (c + 1) * q, :
                    ].astype(jnp.bfloat16)
                r1[hh, c] = exchange(0, hh, c)
                r1[hh, c].start()

        acc = {}
        r2 = {}
        cp_r.wait()
        cp_g.wait()
        for c in range(C):
            for hh in range(2):
                r1[hh, c].wait()
                comm[slot(1, hh, c, 0), :, :] = (
                    comm[slot(0, hh, c, 0), :, :] + comm[slot(0, hh, c, 1), :, :]
                )
                r2[hh, c] = exchange(1, hh, c)
                r2[hh, c].start()
                row0 = hh * h + c * q
                acc[hh, c] = (
                    xv[hh, c * q : (c + 1) * q, :]
                    + comm[slot(0, hh, c, 1), :, :].astype(jnp.float32)
                    + rv[row0 : row0 + q, :]
                )

        gam = jnp.reshape(gv[...], (1, d))

        for c in range(C):
            for hh in range(2):
                r2[hh, c].wait()
                row0 = hh * h + c * q
                y = acc[hh, c] + comm[slot(1, hh, c, 1), :, :].astype(jnp.float32)
                rms = jnp.sqrt(jnp.mean(y * y, axis=-1, keepdims=True) + 1e-6)
                out_ref[row0 : row0 + q, :] = y / rms * gam

    return pl.pallas_call(
        body,
        out_shape=jax.ShapeDtypeStruct((m, d), jnp.float32),
        in_specs=[
            pl.BlockSpec(memory_space=pl.ANY),
            pl.BlockSpec(memory_space=pl.ANY),
            pl.BlockSpec(memory_space=pl.ANY),
        ],
        out_specs=pl.BlockSpec(memory_space=pltpu.VMEM),
        scratch_shapes=[
            pltpu.VMEM((2, h, d), jnp.float32),
            pltpu.VMEM((m, d), jnp.float32),
            pltpu.VMEM((d,), jnp.float32),
            pltpu.VMEM((4 * 2 * C, q, d), jnp.bfloat16),
            pltpu.SemaphoreType.DMA((2 * 2 * C,)),
            pltpu.SemaphoreType.DMA((2 * 2 * C,)),
            pltpu.SemaphoreType.DMA((2 * C + 2,)),
        ],
        compiler_params=pltpu.CompilerParams(collective_id=0),
    )(
        pltpu.with_memory_space_constraint(partial, pltpu.MemorySpace.HBM),
        pltpu.with_memory_space_constraint(resid, pltpu.MemorySpace.HBM),
        pltpu.with_memory_space_constraint(gamma, pltpu.MemorySpace.HBM),
    )
